# baseline (speedup 1.0000x reference)
import functools
import math
import jax
import jax.numpy as jnp
from jax import lax
from jax.experimental import pallas as pl
from jax.experimental.pallas import tpu as pltpu
from jax.experimental.pallas import tpu_sc as plsc

N = 10000
FIN = 16
C = 128
K = 16
NB = 2
NL = 4

NP = 10240
RB = 256
DEPTH = 4

_NEG2 = -2.0
_BIG = 1e30


def _knn_kernel(xr_ref, xt_ref, sqc_ref, sqr_ref, out_ref, g_scr,
                sv_scr, si_scr, o_scr):
    g_scr[...] = jnp.dot(xr_ref[0], xt_ref[0],
                         preferred_element_type=jnp.float32)
    rowbase = pl.program_id(1) * RB

    lane = jax.lax.broadcasted_iota(jnp.int32, (8, 128), 1)
    subl = jax.lax.broadcasted_iota(jnp.int32, (8, 128), 0)

    for s in range(RB // 8):
        off = (8 * s) % 128
        cst = rowbase + 8 * s - off
        r0 = pl.ds(8 * s, 8)
        cs = pl.ds(cst, 128)
        g_scr[r0, cs] = jnp.where(lane == off + subl, _BIG, g_scr[r0, cs])

    U = 8
    IBIG = jnp.int32(2 ** 30)

    def strips_body(si, _):
        base = si * U
        sqr = [jnp.broadcast_to(
            sqr_ref[0, pl.ds((base + u) * 8, 8), 0:1], (8, 128))
            for u in range(U)]

        def chunk_body(g, S):
            sqg = jnp.broadcast_to(sqc_ref[0, 0:1, pl.ds(g * 128, 128)],
                                   (8, 128))
            ci0 = lane + g * 128
            out = []
            for u in range(U):
                s0, s1, s2, s3, i0, i1, i2, i3 = S[u * 8:(u + 1) * 8]
                gv = g_scr[pl.ds((base + u) * 8, 8), pl.ds(g * 128, 128)]
                d = (sqr[u] + sqg) + gv
                ci = ci0
                c = d < s0
                s0, d = jnp.minimum(s0, d), jnp.maximum(s0, d)
                i0, ci = jnp.where(c, ci, i0), jnp.where(c, i0, ci)
                c = d < s1
                s1, d = jnp.minimum(s1, d), jnp.maximum(s1, d)
                i1, ci = jnp.where(c, ci, i1), jnp.where(c, i1, ci)
                c = d < s2
                s2, d = jnp.minimum(s2, d), jnp.maximum(s2, d)
                i2, ci = jnp.where(c, ci, i2), jnp.where(c, i2, ci)
                c = d < s3
                s3, i3 = jnp.minimum(s3, d), jnp.where(c, ci, i3)
                out.extend((s0, s1, s2, s3, i0, i1, i2, i3))
            return tuple(out)

        big = jnp.full((8, 128), _BIG, jnp.float32)
        zi = jnp.zeros((8, 128), jnp.int32)
        S = jax.lax.fori_loop(0, NP // 128, chunk_body,
                              (big, big, big, big, zi, zi, zi, zi) * U)
        for u in range(U):
            r0 = pl.ds((base + u) * 8, 8)
            for j in range(DEPTH):
                sv_scr[j, r0, :] = S[u * 8 + j]
                si_scr[j, r0, :] = S[u * 8 + DEPTH + j]
        return 0

    jax.lax.fori_loop(0, RB // 8 // U, strips_body, 0)

    def k_body(k, _):
        for strip in range(RB // 8):
            r0 = pl.ds(strip * 8, 8)
            s0, s1 = sv_scr[0, r0, :], sv_scr[1, r0, :]
            s2, s3 = sv_scr[2, r0, :], sv_scr[3, r0, :]
            i0, i1 = si_scr[0, r0, :], si_scr[1, r0, :]
            i2, i3 = si_scr[2, r0, :], si_scr[3, r0, :]
            rmin = jnp.min(s0, axis=1, keepdims=True)
            wmask = s0 == rmin
            wi = jnp.min(jnp.where(wmask, i0, IBIG), axis=1, keepdims=True)
            o_scr[r0, :] = jnp.where(lane == k,
                                     jnp.broadcast_to(wi, (8, 128)),
                                     o_scr[r0, :])
            win = wmask & (i0 == wi)
            sv_scr[0, r0, :] = jnp.where(win, s1, s0)
            sv_scr[1, r0, :] = jnp.where(win, s2, s1)
            sv_scr[2, r0, :] = jnp.where(win, s3, s2)
            sv_scr[3, r0, :] = jnp.where(win, _BIG, s3)
            si_scr[0, r0, :] = jnp.where(win, i1, i0)
            si_scr[1, r0, :] = jnp.where(win, i2, i1)
            si_scr[2, r0, :] = jnp.where(win, i3, i2)
        return 0

    jax.lax.fori_loop(0, K, k_body, 0)
    for strip in range(RB // 8):
        r0 = pl.ds(strip * 8, 8)
        out_ref[0, r0, :] = o_scr[r0, :K]


def _knn_pallas(xpad, xt, sqc, sqr):
    return pl.pallas_call(
        _knn_kernel,
        grid=(NB, NP // RB),
        in_specs=[
            pl.BlockSpec((1, RB, C), lambda b, i: (b, i, 0)),
            pl.BlockSpec((1, C, NP), lambda b, i: (b, 0, 0)),
            pl.BlockSpec((1, 1, NP), lambda b, i: (b, 0, 0)),
            pl.BlockSpec((1, RB, 1), lambda b, i: (b, i, 0)),
        ],
        out_specs=pl.BlockSpec((1, RB, K), lambda b, i: (b, i, 0)),
        out_shape=jax.ShapeDtypeStruct((NB, NP, K), jnp.int32),
        scratch_shapes=[pltpu.VMEM((RB, NP), jnp.float32),
                        pltpu.VMEM((DEPTH, RB, 128), jnp.float32),
                        pltpu.VMEM((DEPTH, RB, 128), jnp.int32),
                        pltpu.VMEM((RB, 128), jnp.int32)],
    )(xpad, xt, sqc, sqr)


_MM_BLK = 1000


def _xlr_kernel(cur_ref, w_ref, out_ref):
    out_ref[...] = jnp.dot(cur_ref[...], w_ref[...],
                           preferred_element_type=jnp.float32)


def _xlr_pallas(cur, w2):
    return pl.pallas_call(
        _xlr_kernel,
        grid=(NB * N // _MM_BLK,),
        in_specs=[
            pl.BlockSpec((_MM_BLK, C), lambda i: (i, 0)),
            pl.BlockSpec((C, 2 * C), lambda i: (0, 0)),
        ],
        out_specs=pl.BlockSpec((_MM_BLK, 2 * C), lambda i: (i, 0)),
        out_shape=jax.ShapeDtypeStruct((NB * N, 2 * C), jnp.float32),
    )(cur, w2)


_GB = NB * N * K
_GCH = 400


def _sc_gather(table, idxf):
    info = plsc.get_sparse_core_info()
    nc, ns = info.num_cores, info.num_subcores
    nw = nc * ns
    bpw = _GB // nw
    mesh = plsc.VectorSubcoreMesh(core_axis_name="c", subcore_axis_name="s")

    @functools.partial(
        pl.kernel, mesh=mesh,
        out_type=jax.ShapeDtypeStruct((_GB, C), jnp.float32),
        scratch_types=[
            pltpu.VMEM((_GCH,), jnp.int32),
            pltpu.VMEM((_GCH, C), jnp.float32),
            pltpu.VMEM((_GCH,), jnp.int32),
            pltpu.VMEM((_GCH, C), jnp.float32),
            pltpu.SemaphoreType.DMA,
            pltpu.SemaphoreType.DMA,
        ],
    )
    def gk(table_hbm, idx_hbm, out_hbm, idx_v0, rows_v0, idx_v1, rows_v1,
           sem0, sem1):
        wid = lax.axis_index("s") * nc + lax.axis_index("c")
        base = wid * bpw
        idx_v = (idx_v0, idx_v1)
        rows_v = (rows_v0, rows_v1)
        sem = (sem0, sem1)
        nch = bpw // _GCH
        cps = [None, None]
        for ci in range(nch + 1):
            if ci < nch:
                p = ci % 2
                off = base + ci * _GCH
                pltpu.sync_copy(idx_hbm.at[pl.ds(off, _GCH)], idx_v[p])
                cps[p] = pltpu.async_copy(table_hbm.at[idx_v[p]], rows_v[p],
                                          sem[p])
            if ci > 0:
                q = (ci - 1) % 2
                cps[q].wait()
                pltpu.sync_copy(rows_v[q],
                                out_hbm.at[pl.ds(base + (ci - 1) * _GCH, _GCH)])

    return gk(table, idxf)


_AT_BLK = 400


def _attn_kernel(xj_ref, xr_ref, att_ref, bias_ref, out_ref, sq_ref):
    xrb = xr_ref[...]
    es = []
    for k in range(K):
        t = xj_ref[k] + xrb
        lk = jnp.where(t >= 0, t, 0.2 * t)
        es.append(jnp.dot(lk, att_ref[...],
                          preferred_element_type=jnp.float32))
    e = jnp.concatenate(es, axis=1)
    m = jnp.max(e, axis=1, keepdims=True)
    p = jnp.exp(e - m)
    a = p / jnp.sum(p, axis=1, keepdims=True)
    t = [a[:, k:k + 1] * xj_ref[k] for k in range(K)]
    for step in (8, 4, 2, 1):
        t = [t[i] + t[i + step] for i in range(step)]
    cur = jnp.maximum(t[0] + bias_ref[...], 0.0)
    out_ref[...] = cur
    sq_ref[...] = jnp.sum(cur * cur, axis=1, keepdims=True)


def _attn_pallas(xj, xr, attl, biasl):
    return pl.pallas_call(
        _attn_kernel,
        grid=(NB * N // _AT_BLK,),
        in_specs=[
            pl.BlockSpec((K, _AT_BLK, C), lambda i: (0, i, 0)),
            pl.BlockSpec((_AT_BLK, C), lambda i: (i, 0)),
            pl.BlockSpec((C, 1), lambda i: (0, 0)),
            pl.BlockSpec((1, C), lambda i: (0, 0)),
        ],
        out_specs=[
            pl.BlockSpec((_AT_BLK, C), lambda i: (i, 0)),
            pl.BlockSpec((_AT_BLK, 1), lambda i: (i, 0)),
        ],
        out_shape=[jax.ShapeDtypeStruct((NB * N, C), jnp.float32),
                   jax.ShapeDtypeStruct((NB * N, 1), jnp.float32)],
    )(xj, xr, attl[:, None], biasl[None, :])


_HEAD_BLK = 1000


def _head_kernel(o0_ref, o1_ref, o2_ref, o3_ref, o4_ref, w1_ref, b1_ref,
                 w2_ref, b2_ref, wg_ref, bg_ref, geod_ref, alpha_ref,
                 out_ref):
    acc = jnp.broadcast_to(b1_ref[...], (_HEAD_BLK, 256))
    for c, o_ref in enumerate((o0_ref, o1_ref, o2_ref, o3_ref, o4_ref)):
        mc = jnp.maximum(o_ref[0], o_ref[1])
        acc = acc + jnp.dot(mc, w1_ref[pl.ds(c * C, C), :],
                            preferred_element_type=jnp.float32)
    h1 = jnp.maximum(acc, 0.0)
    h2 = jnp.maximum(jnp.dot(h1, w2_ref[...],
                             preferred_element_type=jnp.float32) + b2_ref[...], 0.0)
    o = jnp.dot(h2, wg_ref[...], preferred_element_type=jnp.float32) + bg_ref[...]
    out_ref[...] = o * jnp.tanh(alpha_ref[...] * geod_ref[...])


def _mlp_head(outs, W1, b1, W2, b2, Wg, bg, geod, alpha):
    specs = [pl.BlockSpec((NB, _HEAD_BLK, C), lambda i: (0, i, 0))
             for _ in range(NL + 1)]
    return pl.pallas_call(
        _head_kernel,
        grid=(N // _HEAD_BLK,),
        in_specs=specs + [
            pl.BlockSpec((C * (NL + 1), 256), lambda i: (0, 0)),
            pl.BlockSpec((1, 256), lambda i: (0, 0)),
            pl.BlockSpec((256, 64), lambda i: (0, 0)),
            pl.BlockSpec((1, 64), lambda i: (0, 0)),
            pl.BlockSpec((64, 3), lambda i: (0, 0)),
            pl.BlockSpec((1, 3), lambda i: (0, 0)),
            pl.BlockSpec((_HEAD_BLK, 1), lambda i: (i, 0)),
            pl.BlockSpec((1, 1), lambda i: (0, 0)),
        ],
        out_specs=pl.BlockSpec((_HEAD_BLK, 3), lambda i: (i, 0)),
        out_shape=jax.ShapeDtypeStruct((N, 3), jnp.float32),
    )(*outs, W1, b1[None, :], W2, b2[None, :], Wg, bg[None, :],
      geod[:, None], alpha[None, None])


def kernel(x, ft_W, ft_b, Wl, Wr, att, bias, W1, b1, W2, b2, Wg, bg, geod, alpha):
    parts = [jax.nn.sigmoid(x[:, i * 8:(i + 1) * 8] @ ft_W[i] + ft_b[i]) for i in range(NB)]
    cur = jnp.concatenate(parts, axis=0)
    cur3 = cur.reshape(NB, N, C)
    sq3 = jnp.sum(cur3 * cur3, axis=2)[..., None]
    outs = [cur3]
    boff = jnp.arange(NB, dtype=jnp.int32)[:, None, None] * N
    for l in range(NL):
        cb = cur.reshape(NB, N, C)
        xpad = jnp.pad(cb, ((0, 0), (0, NP - N), (0, 0)))
        xt = (xpad * _NEG2).transpose(0, 2, 1)
        sqc = jnp.pad(sq3[:, :, 0], ((0, 0), (0, NP - N)),
                      constant_values=_BIG)[:, None, :]
        sqr = sqc.transpose(0, 2, 1)
        idxp = _knn_pallas(xpad, xt, sqc, sqr)
        idxf = (idxp[:, :N, :] + boff).transpose(2, 0, 1).reshape(_GB)
        xlr = _xlr_pallas(cur, jnp.concatenate([Wl[l], Wr[l]], axis=1))
        xl = xlr[:, :C]
        xr = xlr[:, C:]
        xj = _sc_gather(xl, idxf).reshape(K, NB * N, C)
        cur, sqv = _attn_pallas(xj, xr, att[l], bias[l])
        sq3 = sqv.reshape(NB, N, 1)
        outs.append(cur.reshape(NB, N, C))
    return _mlp_head(outs, W1, b1, W2, b2, Wg, bg, geod,
                     jnp.asarray(alpha, jnp.float32))


# speedup vs baseline: 1.0659x; 1.0659x over previous
import functools
import math
import jax
import jax.numpy as jnp
from jax import lax
from jax.experimental import pallas as pl
from jax.experimental.pallas import tpu as pltpu
from jax.experimental.pallas import tpu_sc as plsc

N = 10000
FIN = 16
C = 128
K = 16
NB = 2
NL = 4

NP = 10240
RB = 512
DEPTH = 4

_NEG2 = -2.0
_BIG = 1e30


def _knn_kernel(xr_ref, xt_ref, sqc_ref, sqr_ref, out_ref, g_scr,
                sv_scr, si_scr, o_scr):
    g_scr[...] = jnp.dot(xr_ref[0], xt_ref[0],
                         preferred_element_type=jnp.float32)
    rowbase = pl.program_id(1) * RB

    lane = jax.lax.broadcasted_iota(jnp.int32, (8, 128), 1)
    subl = jax.lax.broadcasted_iota(jnp.int32, (8, 128), 0)

    for s in range(RB // 8):
        off = (8 * s) % 128
        cst = rowbase + 8 * s - off
        r0 = pl.ds(8 * s, 8)
        cs = pl.ds(cst, 128)
        g_scr[r0, cs] = jnp.where(lane == off + subl, _BIG, g_scr[r0, cs])

    U = 8
    IBIG = jnp.int32(2 ** 30)

    def strips_body(si, _):
        base = si * U
        sqr = [jnp.broadcast_to(
            sqr_ref[0, pl.ds((base + u) * 8, 8), 0:1], (8, 128))
            for u in range(U)]

        def chunk_body(g, S):
            sqg = jnp.broadcast_to(sqc_ref[0, 0:1, pl.ds(g * 128, 128)],
                                   (8, 128))
            ci0 = lane + g * 128
            out = []
            for u in range(U):
                s0, s1, s2, s3, i0, i1, i2, i3 = S[u * 8:(u + 1) * 8]
                gv = g_scr[pl.ds((base + u) * 8, 8), pl.ds(g * 128, 128)]
                d = (sqr[u] + sqg) + gv
                ci = ci0
                c = d < s0
                s0, d = jnp.minimum(s0, d), jnp.maximum(s0, d)
                i0, ci = jnp.where(c, ci, i0), jnp.where(c, i0, ci)
                c = d < s1
                s1, d = jnp.minimum(s1, d), jnp.maximum(s1, d)
                i1, ci = jnp.where(c, ci, i1), jnp.where(c, i1, ci)
                c = d < s2
                s2, d = jnp.minimum(s2, d), jnp.maximum(s2, d)
                i2, ci = jnp.where(c, ci, i2), jnp.where(c, i2, ci)
                c = d < s3
                s3, i3 = jnp.minimum(s3, d), jnp.where(c, ci, i3)
                out.extend((s0, s1, s2, s3, i0, i1, i2, i3))
            return tuple(out)

        big = jnp.full((8, 128), _BIG, jnp.float32)
        zi = jnp.zeros((8, 128), jnp.int32)
        S = jax.lax.fori_loop(0, NP // 128, chunk_body,
                              (big, big, big, big, zi, zi, zi, zi) * U)
        for u in range(U):
            r0 = pl.ds((base + u) * 8, 8)
            for j in range(DEPTH):
                sv_scr[j, r0, :] = S[u * 8 + j]
                si_scr[j, r0, :] = S[u * 8 + DEPTH + j]
        return 0

    jax.lax.fori_loop(0, RB // 8 // U, strips_body, 0)

    def k_body(k, _):
        for strip in range(RB // 8):
            r0 = pl.ds(strip * 8, 8)
            s0, s1 = sv_scr[0, r0, :], sv_scr[1, r0, :]
            s2, s3 = sv_scr[2, r0, :], sv_scr[3, r0, :]
            i0, i1 = si_scr[0, r0, :], si_scr[1, r0, :]
            i2, i3 = si_scr[2, r0, :], si_scr[3, r0, :]
            rmin = jnp.min(s0, axis=1, keepdims=True)
            wmask = s0 == rmin
            wi = jnp.min(jnp.where(wmask, i0, IBIG), axis=1, keepdims=True)
            o_scr[r0, :] = jnp.where(lane == k,
                                     jnp.broadcast_to(wi, (8, 128)),
                                     o_scr[r0, :])
            win = wmask & (i0 == wi)
            sv_scr[0, r0, :] = jnp.where(win, s1, s0)
            sv_scr[1, r0, :] = jnp.where(win, s2, s1)
            sv_scr[2, r0, :] = jnp.where(win, s3, s2)
            sv_scr[3, r0, :] = jnp.where(win, _BIG, s3)
            si_scr[0, r0, :] = jnp.where(win, i1, i0)
            si_scr[1, r0, :] = jnp.where(win, i2, i1)
            si_scr[2, r0, :] = jnp.where(win, i3, i2)
        return 0

    jax.lax.fori_loop(0, K, k_body, 0)
    for strip in range(RB // 8):
        r0 = pl.ds(strip * 8, 8)
        out_ref[0, r0, :] = o_scr[r0, :K]


def _knn_pallas(xpad, xt, sqc, sqr):
    return pl.pallas_call(
        _knn_kernel,
        grid=(NB, NP // RB),
        in_specs=[
            pl.BlockSpec((1, RB, C), lambda b, i: (b, i, 0)),
            pl.BlockSpec((1, C, NP), lambda b, i: (b, 0, 0)),
            pl.BlockSpec((1, 1, NP), lambda b, i: (b, 0, 0)),
            pl.BlockSpec((1, RB, 1), lambda b, i: (b, i, 0)),
        ],
        out_specs=pl.BlockSpec((1, RB, K), lambda b, i: (b, i, 0)),
        out_shape=jax.ShapeDtypeStruct((NB, NP, K), jnp.int32),
        scratch_shapes=[pltpu.VMEM((RB, NP), jnp.float32),
                        pltpu.VMEM((DEPTH, RB, 128), jnp.float32),
                        pltpu.VMEM((DEPTH, RB, 128), jnp.int32),
                        pltpu.VMEM((RB, 128), jnp.int32)],
    )(xpad, xt, sqc, sqr)


_MM_BLK = 1000


def _xlr_kernel(cur_ref, w_ref, out_ref):
    out_ref[...] = jnp.dot(cur_ref[...], w_ref[...],
                           preferred_element_type=jnp.float32)


def _xlr_pallas(cur, w2):
    return pl.pallas_call(
        _xlr_kernel,
        grid=(NB * N // _MM_BLK,),
        in_specs=[
            pl.BlockSpec((_MM_BLK, C), lambda i: (i, 0)),
            pl.BlockSpec((C, 2 * C), lambda i: (0, 0)),
        ],
        out_specs=pl.BlockSpec((_MM_BLK, 2 * C), lambda i: (i, 0)),
        out_shape=jax.ShapeDtypeStruct((NB * N, 2 * C), jnp.float32),
    )(cur, w2)


_GB = NB * N * K
_GCH = 400


def _sc_gather(table, idxf):
    info = plsc.get_sparse_core_info()
    nc, ns = info.num_cores, info.num_subcores
    nw = nc * ns
    bpw = _GB // nw
    mesh = plsc.VectorSubcoreMesh(core_axis_name="c", subcore_axis_name="s")

    @functools.partial(
        pl.kernel, mesh=mesh,
        out_type=jax.ShapeDtypeStruct((_GB, C), jnp.float32),
        scratch_types=[
            pltpu.VMEM((_GCH,), jnp.int32),
            pltpu.VMEM((_GCH, C), jnp.float32),
            pltpu.VMEM((_GCH,), jnp.int32),
            pltpu.VMEM((_GCH, C), jnp.float32),
            pltpu.SemaphoreType.DMA,
            pltpu.SemaphoreType.DMA,
        ],
    )
    def gk(table_hbm, idx_hbm, out_hbm, idx_v0, rows_v0, idx_v1, rows_v1,
           sem0, sem1):
        wid = lax.axis_index("s") * nc + lax.axis_index("c")
        base = wid * bpw
        idx_v = (idx_v0, idx_v1)
        rows_v = (rows_v0, rows_v1)
        sem = (sem0, sem1)
        nch = bpw // _GCH
        cps = [None, None]
        for ci in range(nch + 1):
            if ci < nch:
                p = ci % 2
                off = base + ci * _GCH
                pltpu.sync_copy(idx_hbm.at[pl.ds(off, _GCH)], idx_v[p])
                cps[p] = pltpu.async_copy(table_hbm.at[idx_v[p]], rows_v[p],
                                          sem[p])
            if ci > 0:
                q = (ci - 1) % 2
                cps[q].wait()
                pltpu.sync_copy(rows_v[q],
                                out_hbm.at[pl.ds(base + (ci - 1) * _GCH, _GCH)])

    return gk(table, idxf)


_AT_BLK = 400


def _attn_kernel(xj_ref, xr_ref, att_ref, bias_ref, out_ref, sq_ref):
    xrb = xr_ref[...]
    es = []
    for k in range(K):
        t = xj_ref[k] + xrb
        lk = jnp.where(t >= 0, t, 0.2 * t)
        es.append(jnp.dot(lk, att_ref[...],
                          preferred_element_type=jnp.float32))
    e = jnp.concatenate(es, axis=1)
    m = jnp.max(e, axis=1, keepdims=True)
    p = jnp.exp(e - m)
    a = p / jnp.sum(p, axis=1, keepdims=True)
    t = [a[:, k:k + 1] * xj_ref[k] for k in range(K)]
    for step in (8, 4, 2, 1):
        t = [t[i] + t[i + step] for i in range(step)]
    cur = jnp.maximum(t[0] + bias_ref[...], 0.0)
    out_ref[...] = cur
    sq_ref[...] = jnp.sum(cur * cur, axis=1, keepdims=True)


def _attn_pallas(xj, xr, attl, biasl):
    return pl.pallas_call(
        _attn_kernel,
        grid=(NB * N // _AT_BLK,),
        in_specs=[
            pl.BlockSpec((K, _AT_BLK, C), lambda i: (0, i, 0)),
            pl.BlockSpec((_AT_BLK, C), lambda i: (i, 0)),
            pl.BlockSpec((C, 1), lambda i: (0, 0)),
            pl.BlockSpec((1, C), lambda i: (0, 0)),
        ],
        out_specs=[
            pl.BlockSpec((_AT_BLK, C), lambda i: (i, 0)),
            pl.BlockSpec((_AT_BLK, 1), lambda i: (i, 0)),
        ],
        out_shape=[jax.ShapeDtypeStruct((NB * N, C), jnp.float32),
                   jax.ShapeDtypeStruct((NB * N, 1), jnp.float32)],
    )(xj, xr, attl[:, None], biasl[None, :])


_HEAD_BLK = 1000


def _head_kernel(o0_ref, o1_ref, o2_ref, o3_ref, o4_ref, w1_ref, b1_ref,
                 w2_ref, b2_ref, wg_ref, bg_ref, geod_ref, alpha_ref,
                 out_ref):
    acc = jnp.broadcast_to(b1_ref[...], (_HEAD_BLK, 256))
    for c, o_ref in enumerate((o0_ref, o1_ref, o2_ref, o3_ref, o4_ref)):
        mc = jnp.maximum(o_ref[0], o_ref[1])
        acc = acc + jnp.dot(mc, w1_ref[pl.ds(c * C, C), :],
                            preferred_element_type=jnp.float32)
    h1 = jnp.maximum(acc, 0.0)
    h2 = jnp.maximum(jnp.dot(h1, w2_ref[...],
                             preferred_element_type=jnp.float32) + b2_ref[...], 0.0)
    o = jnp.dot(h2, wg_ref[...], preferred_element_type=jnp.float32) + bg_ref[...]
    out_ref[...] = o * jnp.tanh(alpha_ref[...] * geod_ref[...])


def _mlp_head(outs, W1, b1, W2, b2, Wg, bg, geod, alpha):
    specs = [pl.BlockSpec((NB, _HEAD_BLK, C), lambda i: (0, i, 0))
             for _ in range(NL + 1)]
    return pl.pallas_call(
        _head_kernel,
        grid=(N // _HEAD_BLK,),
        in_specs=specs + [
            pl.BlockSpec((C * (NL + 1), 256), lambda i: (0, 0)),
            pl.BlockSpec((1, 256), lambda i: (0, 0)),
            pl.BlockSpec((256, 64), lambda i: (0, 0)),
            pl.BlockSpec((1, 64), lambda i: (0, 0)),
            pl.BlockSpec((64, 3), lambda i: (0, 0)),
            pl.BlockSpec((1, 3), lambda i: (0, 0)),
            pl.BlockSpec((_HEAD_BLK, 1), lambda i: (i, 0)),
            pl.BlockSpec((1, 1), lambda i: (0, 0)),
        ],
        out_specs=pl.BlockSpec((_HEAD_BLK, 3), lambda i: (i, 0)),
        out_shape=jax.ShapeDtypeStruct((N, 3), jnp.float32),
    )(*outs, W1, b1[None, :], W2, b2[None, :], Wg, bg[None, :],
      geod[:, None], alpha[None, None])


def kernel(x, ft_W, ft_b, Wl, Wr, att, bias, W1, b1, W2, b2, Wg, bg, geod, alpha):
    parts = [jax.nn.sigmoid(x[:, i * 8:(i + 1) * 8] @ ft_W[i] + ft_b[i]) for i in range(NB)]
    cur = jnp.concatenate(parts, axis=0)
    cur3 = cur.reshape(NB, N, C)
    sq3 = jnp.sum(cur3 * cur3, axis=2)[..., None]
    outs = [cur3]
    boff = jnp.arange(NB, dtype=jnp.int32)[:, None, None] * N
    for l in range(NL):
        cb = cur.reshape(NB, N, C)
        xpad = jnp.pad(cb, ((0, 0), (0, NP - N), (0, 0)))
        xt = (xpad * _NEG2).transpose(0, 2, 1)
        sqc = jnp.pad(sq3[:, :, 0], ((0, 0), (0, NP - N)),
                      constant_values=_BIG)[:, None, :]
        sqr = sqc.transpose(0, 2, 1)
        idxp = _knn_pallas(xpad, xt, sqc, sqr)
        idxf = (idxp[:, :N, :] + boff).transpose(2, 0, 1).reshape(_GB)
        xlr = _xlr_pallas(cur, jnp.concatenate([Wl[l], Wr[l]], axis=1))
        xl = xlr[:, :C]
        xr = xlr[:, C:]
        xj = _sc_gather(xl, idxf).reshape(K, NB * N, C)
        cur, sqv = _attn_pallas(xj, xr, att[l], bias[l])
        sq3 = sqv.reshape(NB, N, 1)
        outs.append(cur.reshape(NB, N, C))
    return _mlp_head(outs, W1, b1, W2, b2, Wg, bg, geod,
                     jnp.asarray(alpha, jnp.float32))
